# Initial kernel scaffold; baseline (speedup 1.0000x reference)
import jax
import jax.numpy as jnp
from jax.experimental import pallas as pl


def kernel(x, edge_index, W1, b1, W2, b2, Wfc, bfc):
    raise NotImplementedError("write your pallas kernel here")


# speedup vs baseline: 22.0003x; 22.0003x over previous
import functools

import jax
import jax.numpy as jnp
from jax import lax
from jax.experimental import pallas as pl
from jax.experimental.pallas import tpu as pltpu
from jax.experimental.pallas import tpu_sc as plsc

N = 10000
E = 320000
F = 128
L = 16
NC, NS = 2, 16
NW = NC * NS
CH = 128
CPT = 80
EPW = CH * CPT
E_PAD = EPW * NW
PAD = E_PAD - E
N_DUMMY = 112
N_ACC = N + N_DUMMY
RPT = N_ACC // NS

_mesh = lambda: plsc.VectorSubcoreMesh(
    core_axis_name="c", subcore_axis_name="s", num_cores=NC, num_subcores=NS)


def _zero_fill(ref, rows, cols):
    z = jnp.zeros((16,), jnp.float32)

    def body(r, _):
        for k in range(cols // 16):
            ref[r, pl.ds(k * 16, 16)] = z
        return _

    lax.fori_loop(0, rows, body, None)


def _sc_degree(dstP):
    @functools.partial(
        pl.kernel,
        out_type=jax.ShapeDtypeStruct((NC, N_ACC, L), jnp.float32),
        mesh=_mesh(),
        scratch_types=[
            pltpu.VMEM((CPT, CH), jnp.int32),
            pltpu.VMEM((CH, L), jnp.float32),
            pltpu.VMEM_SHARED((N_ACC, L), jnp.float32),
        ],
    )
    def k(dstP_hbm, out_hbm, dst_v, ones_v, acc):
        cid = lax.axis_index("c")
        sid = lax.axis_index("s")
        w = cid * NS + sid
        pltpu.sync_copy(dstP_hbm.at[pl.ds(w * CPT, CPT)], dst_v)
        _zero_fill(ones_v, CH, L)
        base = sid * RPT
        for r in range(0, RPT, CH):
            sz = min(CH, RPT - r)
            pltpu.sync_copy(ones_v.at[pl.ds(0, sz)], acc.at[pl.ds(base + r, sz)])
        one = jnp.ones((16,), jnp.float32)

        def fill(r, _):
            ones_v[r, :] = one
            return _

        lax.fori_loop(0, CH, fill, None)
        plsc.subcore_barrier()

        def body(j, _):
            pltpu.sync_copy(ones_v, acc.at[dst_v.at[j]], add=True)
            return _

        lax.fori_loop(0, CPT, body, None)
        plsc.subcore_barrier()
        pltpu.sync_copy(acc.at[pl.ds(base, RPT)],
                        out_hbm.at[cid, pl.ds(base, RPT)])

    return k(dstP)


def _sc_propagate(srcP, dstP, hs, D):
    @functools.partial(
        pl.kernel,
        out_type=jax.ShapeDtypeStruct((NC, N_ACC, D), jnp.float32),
        mesh=_mesh(),
        scratch_types=[
            pltpu.VMEM((CPT, CH), jnp.int32),
            pltpu.VMEM((CPT, CH), jnp.int32),
            pltpu.VMEM((CH, D), jnp.float32),
            pltpu.VMEM_SHARED((N_ACC, D), jnp.float32),
            pltpu.SemaphoreType.DMA,
        ],
    )
    def k(srcP_hbm, dstP_hbm, hs_hbm, out_hbm, src_v, dst_v, rows, acc, sem):
        cid = lax.axis_index("c")
        sid = lax.axis_index("s")
        w = cid * NS + sid
        pltpu.sync_copy(srcP_hbm.at[pl.ds(w * CPT, CPT)], src_v)
        pltpu.sync_copy(dstP_hbm.at[pl.ds(w * CPT, CPT)], dst_v)
        _zero_fill(rows, CH, D)
        base = sid * RPT
        for r in range(0, RPT, CH):
            sz = min(CH, RPT - r)
            pltpu.sync_copy(rows.at[pl.ds(0, sz)], acc.at[pl.ds(base + r, sz)])
        plsc.subcore_barrier()

        def body(j, _):
            pltpu.async_copy(hs_hbm.at[src_v.at[j]], rows, sem).wait()
            pltpu.sync_copy(rows, acc.at[dst_v.at[j]], add=True)
            return _

        lax.fori_loop(0, CPT, body, None)
        plsc.subcore_barrier()
        pltpu.sync_copy(acc.at[pl.ds(base, RPT)],
                        out_hbm.at[cid, pl.ds(base, RPT)])

    return k(srcP, dstP, hs)


def _tc_k1(x, W1, pdeg):
    R = 1000

    def body(x_ref, w_ref, p_ref, hs_ref, dinv_ref):
        deg = p_ref[0, :, 0:1] + p_ref[1, :, 0:1] + 1.0
        dinv = lax.rsqrt(deg)
        h = jnp.dot(x_ref[...], w_ref[...], preferred_element_type=jnp.float32)
        hs_ref[...] = h * dinv
        dinv_ref[...] = dinv

    return pl.pallas_call(
        body,
        grid=(N // R,),
        in_specs=[
            pl.BlockSpec((R, F), lambda i: (i, 0)),
            pl.BlockSpec((F, F), lambda i: (0, 0)),
            pl.BlockSpec((NC, R, L), lambda i: (0, i, 0)),
        ],
        out_specs=[
            pl.BlockSpec((R, F), lambda i: (i, 0)),
            pl.BlockSpec((R, 1), lambda i: (i, 0)),
        ],
        out_shape=[
            jax.ShapeDtypeStruct((N, F), jnp.float32),
            jax.ShapeDtypeStruct((N, 1), jnp.float32),
        ],
    )(x, W1, pdeg)


def _tc_k2(p1, hs1, dinv, b1):
    R = 1000

    def body(p_ref, hs_ref, dinv_ref, b_ref, hs1o_ref):
        dinv = dinv_ref[...]
        s = p_ref[0] + p_ref[1]
        h1o = jnp.maximum(dinv * s + dinv * hs_ref[...] + b_ref[...], 0.0)
        hs1o_ref[...] = h1o * dinv

    return pl.pallas_call(
        body,
        grid=(N // R,),
        in_specs=[
            pl.BlockSpec((NC, R, F), lambda i: (0, i, 0)),
            pl.BlockSpec((R, F), lambda i: (i, 0)),
            pl.BlockSpec((R, 1), lambda i: (i, 0)),
            pl.BlockSpec((1, F), lambda i: (0, 0)),
        ],
        out_specs=pl.BlockSpec((R, F), lambda i: (i, 0)),
        out_shape=jax.ShapeDtypeStruct((N, F), jnp.float32),
    )(p1, hs1, dinv, b1)


def _tc_k3(p2, hs1o, dinv, W2p, b2p, Wfcp, bfc):
    R = 1000

    def body(p_ref, hs_ref, dinv_ref, w2_ref, b_ref, w_ref, bfc_ref, out_ref):
        dinv = dinv_ref[...]
        s = p_ref[0] + p_ref[1]
        prop = dinv * s + dinv * hs_ref[...]
        h2 = jnp.dot(prop, w2_ref[...], preferred_element_type=jnp.float32)
        a = jnp.maximum(h2 + b_ref[...], 0.0)
        out_ref[...] = (
            jnp.dot(a, w_ref[...], preferred_element_type=jnp.float32)
            + bfc_ref[...])

    return pl.pallas_call(
        body,
        grid=(N // R,),
        in_specs=[
            pl.BlockSpec((NC, R, F), lambda i: (0, i, 0)),
            pl.BlockSpec((R, F), lambda i: (i, 0)),
            pl.BlockSpec((R, 1), lambda i: (i, 0)),
            pl.BlockSpec((F, L), lambda i: (0, 0)),
            pl.BlockSpec((1, L), lambda i: (0, 0)),
            pl.BlockSpec((L, 10), lambda i: (0, 0)),
            pl.BlockSpec((1, 10), lambda i: (0, 0)),
        ],
        out_specs=pl.BlockSpec((R, 10), lambda i: (i, 0)),
        out_shape=jax.ShapeDtypeStruct((N, 10), jnp.float32),
    )(p2, hs1o, dinv, W2p, b2p, Wfcp, bfc)


def kernel(x, edge_index, W1, b1, W2, b2, Wfc, bfc):
    src = edge_index[0].astype(jnp.int32)
    dst = edge_index[1].astype(jnp.int32)
    pidx = jnp.arange(PAD, dtype=jnp.int32)
    srcP = jnp.concatenate([src, pidx % N]).reshape(E_PAD // CH, CH)
    dstP = jnp.concatenate([dst, N + (pidx % N_DUMMY)]).reshape(E_PAD // CH, CH)
    W2p = jnp.pad(W2, ((0, 0), (0, L - W2.shape[1])))
    b2p = jnp.pad(b2, (0, L - b2.shape[0]))[None, :]
    Wfcp = jnp.pad(Wfc, ((0, L - Wfc.shape[0]), (0, 0)))

    pdeg = _sc_degree(dstP)
    hs1, dinv = _tc_k1(x, W1, pdeg)
    p1 = _sc_propagate(srcP, dstP, hs1, F)
    hs1o = _tc_k2(p1, hs1, dinv, b1[None, :])
    p2 = _sc_propagate(srcP, dstP, hs1o, F)
    return _tc_k3(p2, hs1o, dinv, W2p, b2p, Wfcp, bfc[None, :])


# speedup vs baseline: 24.5127x; 1.1142x over previous
import functools

import jax
import jax.numpy as jnp
from jax import lax
from jax.experimental import pallas as pl
from jax.experimental.pallas import tpu as pltpu
from jax.experimental.pallas import tpu_sc as plsc

N = 10000
E = 320000
F = 128
L = 16
NC, NS = 2, 16
NW = NC * NS
CH = 128
CPT = 80
EPW = CH * CPT
E_PAD = EPW * NW
PAD = E_PAD - E
N_DUMMY = 112
N_ACC = N + N_DUMMY
RPT = N_ACC // NS

_mesh = lambda: plsc.VectorSubcoreMesh(
    core_axis_name="c", subcore_axis_name="s", num_cores=NC, num_subcores=NS)


def _zero_fill(ref, rows, cols):
    z = jnp.zeros((16,), jnp.float32)

    def body(r, _):
        for k in range(cols // 16):
            ref[r, pl.ds(k * 16, 16)] = z
        return _

    lax.fori_loop(0, rows, body, None)


def _sc_degree(dstP):
    @functools.partial(
        pl.kernel,
        out_type=jax.ShapeDtypeStruct((NC, N_ACC, L), jnp.float32),
        mesh=_mesh(),
        scratch_types=[
            pltpu.VMEM((CPT, CH), jnp.int32),
            pltpu.VMEM((CH, L), jnp.float32),
            pltpu.VMEM_SHARED((N_ACC, L), jnp.float32),
        ],
    )
    def k(dstP_hbm, out_hbm, dst_v, ones_v, acc):
        cid = lax.axis_index("c")
        sid = lax.axis_index("s")
        w = cid * NS + sid
        pltpu.sync_copy(dstP_hbm.at[pl.ds(w * CPT, CPT)], dst_v)
        _zero_fill(ones_v, CH, L)
        base = sid * RPT
        for r in range(0, RPT, CH):
            sz = min(CH, RPT - r)
            pltpu.sync_copy(ones_v.at[pl.ds(0, sz)], acc.at[pl.ds(base + r, sz)])
        one = jnp.ones((16,), jnp.float32)

        def fill(r, _):
            ones_v[r, :] = one
            return _

        lax.fori_loop(0, CH, fill, None)
        plsc.subcore_barrier()

        def body(j, _):
            pltpu.sync_copy(ones_v, acc.at[dst_v.at[j]], add=True)
            return _

        lax.fori_loop(0, CPT, body, None)
        plsc.subcore_barrier()
        pltpu.sync_copy(acc.at[pl.ds(base, RPT)],
                        out_hbm.at[cid, pl.ds(base, RPT)])

    return k(dstP)


def _sc_propagate(srcP, dstP, hs, D):
    @functools.partial(
        pl.kernel,
        out_type=jax.ShapeDtypeStruct((NC, N_ACC, D), jnp.float32),
        mesh=_mesh(),
        scratch_types=[
            pltpu.VMEM((CPT // 2, CH), jnp.int32),
            pltpu.VMEM((CPT // 2, CH), jnp.int32),
            pltpu.VMEM((CH, D), jnp.float32),
            pltpu.VMEM((CH, D), jnp.float32),
            pltpu.VMEM_SHARED((N_ACC, D), jnp.float32),
            pltpu.SemaphoreType.DMA,
        ],
    )
    def k(srcP_hbm, dstP_hbm, hs_hbm, out_hbm, src_v, dst_v, rows0, rows1,
          acc, sem0):
        cid = lax.axis_index("c")
        sid = lax.axis_index("s")
        w = cid * NS + sid
        IH = CPT // 2
        _zero_fill(rows0, CH, D)
        base = sid * RPT
        for r in range(0, RPT, CH):
            sz = min(CH, RPT - r)
            pltpu.sync_copy(rows0.at[pl.ds(0, sz)], acc.at[pl.ds(base + r, sz)])
        plsc.subcore_barrier()

        for ph in range(2):
            pbase = w * CPT + ph * IH
            pltpu.sync_copy(srcP_hbm.at[pl.ds(pbase, IH)], src_v)
            pltpu.sync_copy(dstP_hbm.at[pl.ds(pbase, IH)], dst_v)

            def body(t, _):
                j = 2 * t
                d0 = pltpu.async_copy(hs_hbm.at[src_v.at[j]], rows0, sem0)
                d0.wait()
                d1 = pltpu.async_copy(hs_hbm.at[src_v.at[j + 1]], rows1, sem0)
                pltpu.sync_copy(rows0, acc.at[dst_v.at[j]], add=True)
                d1.wait()
                pltpu.sync_copy(rows1, acc.at[dst_v.at[j + 1]], add=True)
                return _

            lax.fori_loop(0, IH // 2, body, None)
        plsc.subcore_barrier()
        pltpu.sync_copy(acc.at[pl.ds(base, RPT)],
                        out_hbm.at[cid, pl.ds(base, RPT)])

    return k(srcP, dstP, hs)


def _tc_k1(x, W1, pdeg):
    R = 1000

    def body(x_ref, w_ref, p_ref, hs_ref, dinv_ref):
        deg = p_ref[0, :, 0:1] + p_ref[1, :, 0:1] + 1.0
        dinv = lax.rsqrt(deg)
        h = jnp.dot(x_ref[...], w_ref[...], preferred_element_type=jnp.float32)
        hs_ref[...] = h * dinv
        dinv_ref[...] = dinv

    return pl.pallas_call(
        body,
        grid=(N // R,),
        in_specs=[
            pl.BlockSpec((R, F), lambda i: (i, 0)),
            pl.BlockSpec((F, F), lambda i: (0, 0)),
            pl.BlockSpec((NC, R, L), lambda i: (0, i, 0)),
        ],
        out_specs=[
            pl.BlockSpec((R, F), lambda i: (i, 0)),
            pl.BlockSpec((R, 1), lambda i: (i, 0)),
        ],
        out_shape=[
            jax.ShapeDtypeStruct((N, F), jnp.float32),
            jax.ShapeDtypeStruct((N, 1), jnp.float32),
        ],
    )(x, W1, pdeg)


def _tc_k2(p1, hs1, dinv, b1):
    R = 1000

    def body(p_ref, hs_ref, dinv_ref, b_ref, hs1o_ref):
        dinv = dinv_ref[...]
        s = p_ref[0] + p_ref[1]
        h1o = jnp.maximum(dinv * s + dinv * hs_ref[...] + b_ref[...], 0.0)
        hs1o_ref[...] = h1o * dinv

    return pl.pallas_call(
        body,
        grid=(N // R,),
        in_specs=[
            pl.BlockSpec((NC, R, F), lambda i: (0, i, 0)),
            pl.BlockSpec((R, F), lambda i: (i, 0)),
            pl.BlockSpec((R, 1), lambda i: (i, 0)),
            pl.BlockSpec((1, F), lambda i: (0, 0)),
        ],
        out_specs=pl.BlockSpec((R, F), lambda i: (i, 0)),
        out_shape=jax.ShapeDtypeStruct((N, F), jnp.float32),
    )(p1, hs1, dinv, b1)


def _tc_k3(p2, hs1o, dinv, W2p, b2p, Wfcp, bfc):
    R = 1000

    def body(p_ref, hs_ref, dinv_ref, w2_ref, b_ref, w_ref, bfc_ref, out_ref):
        dinv = dinv_ref[...]
        s = p_ref[0] + p_ref[1]
        prop = dinv * s + dinv * hs_ref[...]
        h2 = jnp.dot(prop, w2_ref[...], preferred_element_type=jnp.float32)
        a = jnp.maximum(h2 + b_ref[...], 0.0)
        out_ref[...] = (
            jnp.dot(a, w_ref[...], preferred_element_type=jnp.float32)
            + bfc_ref[...])

    return pl.pallas_call(
        body,
        grid=(N // R,),
        in_specs=[
            pl.BlockSpec((NC, R, F), lambda i: (0, i, 0)),
            pl.BlockSpec((R, F), lambda i: (i, 0)),
            pl.BlockSpec((R, 1), lambda i: (i, 0)),
            pl.BlockSpec((F, L), lambda i: (0, 0)),
            pl.BlockSpec((1, L), lambda i: (0, 0)),
            pl.BlockSpec((L, 10), lambda i: (0, 0)),
            pl.BlockSpec((1, 10), lambda i: (0, 0)),
        ],
        out_specs=pl.BlockSpec((R, 10), lambda i: (i, 0)),
        out_shape=jax.ShapeDtypeStruct((N, 10), jnp.float32),
    )(p2, hs1o, dinv, W2p, b2p, Wfcp, bfc)


def kernel(x, edge_index, W1, b1, W2, b2, Wfc, bfc):
    src = edge_index[0].astype(jnp.int32)
    dst = edge_index[1].astype(jnp.int32)
    pidx = jnp.arange(PAD, dtype=jnp.int32)
    srcP = jnp.concatenate([src, pidx % N]).reshape(E_PAD // CH, CH)
    dstP = jnp.concatenate([dst, N + (pidx % N_DUMMY)]).reshape(E_PAD // CH, CH)
    W2p = jnp.pad(W2, ((0, 0), (0, L - W2.shape[1])))
    b2p = jnp.pad(b2, (0, L - b2.shape[0]))[None, :]
    Wfcp = jnp.pad(Wfc, ((0, L - Wfc.shape[0]), (0, 0)))

    pdeg = _sc_degree(dstP)
    hs1, dinv = _tc_k1(x, W1, pdeg)
    p1 = _sc_propagate(srcP, dstP, hs1, F)
    hs1o = _tc_k2(p1, hs1, dinv, b1[None, :])
    p2 = _sc_propagate(srcP, dstP, hs1o, F)
    return _tc_k3(p2, hs1o, dinv, W2p, b2p, Wfcp, bfc[None, :])


# speedup vs baseline: 30.5343x; 1.2457x over previous
import functools

import jax
import jax.numpy as jnp
from jax import lax
from jax.experimental import pallas as pl
from jax.experimental.pallas import tpu as pltpu
from jax.experimental.pallas import tpu_sc as plsc

N = 10000
E = 320000
F = 128
L = 16
NC, NS = 2, 16
NW = NC * NS
CH = 128
CPT = 80
EPW = CH * CPT
E_PAD = EPW * NW
PAD = E_PAD - E
N_DUMMY = 112
N_ACC = N + N_DUMMY
RPT = N_ACC // NS

_mesh = lambda: plsc.VectorSubcoreMesh(
    core_axis_name="c", subcore_axis_name="s", num_cores=NC, num_subcores=NS)


def _zero_fill(ref, rows, cols):
    z = jnp.zeros((16,), jnp.float32)

    def body(r, _):
        for k in range(cols // 16):
            ref[r, pl.ds(k * 16, 16)] = z
        return _

    lax.fori_loop(0, rows, body, None)


def _sc_degree(dstP):
    @functools.partial(
        pl.kernel,
        out_type=jax.ShapeDtypeStruct((NC, N_ACC, L), jnp.float32),
        mesh=_mesh(),
        scratch_types=[
            pltpu.VMEM((CPT, CH), jnp.int32),
            pltpu.VMEM((CH, L), jnp.float32),
            pltpu.VMEM_SHARED((N_ACC, L), jnp.float32),
        ],
    )
    def k(dstP_hbm, out_hbm, dst_v, ones_v, acc):
        cid = lax.axis_index("c")
        sid = lax.axis_index("s")
        w = cid * NS + sid
        pltpu.sync_copy(dstP_hbm.at[pl.ds(w * CPT, CPT)], dst_v)
        _zero_fill(ones_v, CH, L)
        base = sid * RPT
        for r in range(0, RPT, CH):
            sz = min(CH, RPT - r)
            pltpu.sync_copy(ones_v.at[pl.ds(0, sz)], acc.at[pl.ds(base + r, sz)])
        one = jnp.ones((16,), jnp.float32)

        def fill(r, _):
            ones_v[r, :] = one
            return _

        lax.fori_loop(0, CH, fill, None)
        plsc.subcore_barrier()

        def body(j, _):
            pltpu.sync_copy(ones_v, acc.at[dst_v.at[j]], add=True)
            return _

        lax.fori_loop(0, CPT, body, None)
        plsc.subcore_barrier()
        pltpu.sync_copy(acc.at[pl.ds(base, RPT)],
                        out_hbm.at[cid, pl.ds(base, RPT)])

    return k(dstP)


def _sc_propagate(srcP, dstP, hs, D):
    @functools.partial(
        pl.kernel,
        out_type=jax.ShapeDtypeStruct((NC, N_ACC, D), jnp.float32),
        mesh=_mesh(),
        scratch_types=[
            pltpu.VMEM((CPT // 2, CH), jnp.int32),
            pltpu.VMEM((CPT // 2, CH), jnp.int32),
            pltpu.VMEM((CH, D), jnp.float32),
            pltpu.VMEM_SHARED((N_ACC, D), jnp.float32),
            pltpu.SemaphoreType.DMA,
        ],
    )
    def k(srcP_hbm, dstP_hbm, hs_hbm, out_hbm, src_v, dst_v, rows0,
          acc, sem0):
        cid = lax.axis_index("c")
        sid = lax.axis_index("s")
        w = cid * NS + sid
        IH = CPT // 2
        _zero_fill(rows0, CH, D)
        base = sid * RPT
        for r in range(0, RPT, CH):
            sz = min(CH, RPT - r)
            pltpu.sync_copy(rows0.at[pl.ds(0, sz)], acc.at[pl.ds(base + r, sz)])
        plsc.subcore_barrier()

        for ph in range(2):
            pbase = w * CPT + ph * IH
            pltpu.sync_copy(srcP_hbm.at[pl.ds(pbase, IH)], src_v)
            pltpu.sync_copy(dstP_hbm.at[pl.ds(pbase, IH)], dst_v)

            def body(j, _):
                pltpu.async_copy(hs_hbm.at[src_v.at[j]], rows0, sem0).wait()
                pltpu.sync_copy(rows0, acc.at[dst_v.at[j]], add=True)
                return _

            lax.fori_loop(0, IH, body, None)
        plsc.subcore_barrier()
        pltpu.sync_copy(acc.at[pl.ds(base, RPT)],
                        out_hbm.at[cid, pl.ds(base, RPT)])

    return k(srcP, dstP, hs)


def _sc_propagate16(srcP, dstP, hs2):
    @functools.partial(
        pl.kernel,
        out_type=jax.ShapeDtypeStruct((NC, N_ACC, L), jnp.float32),
        mesh=_mesh(),
        scratch_types=[
            pltpu.VMEM((CPT // 2, CH), jnp.int32),
            pltpu.VMEM((CPT // 2, CH), jnp.int32),
            pltpu.VMEM((CH, L), jnp.float32),
            pltpu.VMEM_SHARED((N, L), jnp.float32),
            pltpu.VMEM_SHARED((N_ACC, L), jnp.float32),
            pltpu.SemaphoreType.DMA,
        ],
    )
    def k(srcP_hbm, dstP_hbm, hs_hbm, out_hbm, src_v, dst_v, rows0,
          table, acc, sem0):
        cid = lax.axis_index("c")
        sid = lax.axis_index("s")
        w = cid * NS + sid
        IH = CPT // 2
        toff = sid * RPT

        @pl.when(sid < NS - 1)
        def _stage():
            pltpu.sync_copy(hs_hbm.at[pl.ds(toff, RPT)],
                            table.at[pl.ds(toff, RPT)])

        @pl.when(sid == NS - 1)
        def _stage_last():
            pltpu.sync_copy(hs_hbm.at[pl.ds(N - (N - (NS - 1) * RPT),
                                            N - (NS - 1) * RPT)],
                            table.at[pl.ds((NS - 1) * RPT,
                                           N - (NS - 1) * RPT)])
        _zero_fill(rows0, CH, L)
        base = sid * RPT
        for r in range(0, RPT, CH):
            sz = min(CH, RPT - r)
            pltpu.sync_copy(rows0.at[pl.ds(0, sz)], acc.at[pl.ds(base + r, sz)])
        plsc.subcore_barrier()

        for ph in range(2):
            pbase = w * CPT + ph * IH
            pltpu.sync_copy(srcP_hbm.at[pl.ds(pbase, IH)], src_v)
            pltpu.sync_copy(dstP_hbm.at[pl.ds(pbase, IH)], dst_v)

            def body(j, _):
                pltpu.async_copy(table.at[src_v.at[j]], rows0, sem0).wait()
                pltpu.sync_copy(rows0, acc.at[dst_v.at[j]], add=True)
                return _

            lax.fori_loop(0, IH, body, None)
        plsc.subcore_barrier()
        pltpu.sync_copy(acc.at[pl.ds(base, RPT)],
                        out_hbm.at[cid, pl.ds(base, RPT)])

    return k(srcP, dstP, hs2)


def _tc_k1(x, W1, pdeg):
    R = 1000

    def body(x_ref, w_ref, p_ref, hs_ref, dinv_ref):
        deg = p_ref[0, :, 0:1] + p_ref[1, :, 0:1] + 1.0
        dinv = lax.rsqrt(deg)
        h = jnp.dot(x_ref[...], w_ref[...], preferred_element_type=jnp.float32)
        hs_ref[...] = h * dinv
        dinv_ref[...] = dinv

    return pl.pallas_call(
        body,
        grid=(N // R,),
        in_specs=[
            pl.BlockSpec((R, F), lambda i: (i, 0)),
            pl.BlockSpec((F, F), lambda i: (0, 0)),
            pl.BlockSpec((NC, R, L), lambda i: (0, i, 0)),
        ],
        out_specs=[
            pl.BlockSpec((R, F), lambda i: (i, 0)),
            pl.BlockSpec((R, 1), lambda i: (i, 0)),
        ],
        out_shape=[
            jax.ShapeDtypeStruct((N, F), jnp.float32),
            jax.ShapeDtypeStruct((N, 1), jnp.float32),
        ],
    )(x, W1, pdeg)


def _tc_k2(p1, hs1, dinv, b1, W2p):
    R = 1000

    def body(p_ref, hs_ref, dinv_ref, b_ref, w_ref, hs2_ref):
        dinv = dinv_ref[...]
        s = p_ref[0] + p_ref[1]
        h1o = jnp.maximum(dinv * s + dinv * hs_ref[...] + b_ref[...], 0.0)
        h2 = jnp.dot(h1o, w_ref[...], preferred_element_type=jnp.float32)
        hs2_ref[...] = h2 * dinv

    return pl.pallas_call(
        body,
        grid=(N // R,),
        in_specs=[
            pl.BlockSpec((NC, R, F), lambda i: (0, i, 0)),
            pl.BlockSpec((R, F), lambda i: (i, 0)),
            pl.BlockSpec((R, 1), lambda i: (i, 0)),
            pl.BlockSpec((1, F), lambda i: (0, 0)),
            pl.BlockSpec((F, L), lambda i: (0, 0)),
        ],
        out_specs=pl.BlockSpec((R, L), lambda i: (i, 0)),
        out_shape=jax.ShapeDtypeStruct((N, L), jnp.float32),
    )(p1, hs1, dinv, b1, W2p)


def _tc_k3(p2, hs2, dinv, b2p, Wfcp, bfc):
    R = 1000

    def body(p_ref, hs_ref, dinv_ref, b_ref, w_ref, bfc_ref, out_ref):
        dinv = dinv_ref[...]
        s = p_ref[0] + p_ref[1]
        a = jnp.maximum(dinv * s + dinv * hs_ref[...] + b_ref[...], 0.0)
        out_ref[...] = (
            jnp.dot(a, w_ref[...], preferred_element_type=jnp.float32)
            + bfc_ref[...])

    return pl.pallas_call(
        body,
        grid=(N // R,),
        in_specs=[
            pl.BlockSpec((NC, R, L), lambda i: (0, i, 0)),
            pl.BlockSpec((R, L), lambda i: (i, 0)),
            pl.BlockSpec((R, 1), lambda i: (i, 0)),
            pl.BlockSpec((1, L), lambda i: (0, 0)),
            pl.BlockSpec((L, 10), lambda i: (0, 0)),
            pl.BlockSpec((1, 10), lambda i: (0, 0)),
        ],
        out_specs=pl.BlockSpec((R, 10), lambda i: (i, 0)),
        out_shape=jax.ShapeDtypeStruct((N, 10), jnp.float32),
    )(p2, hs2, dinv, b2p, Wfcp, bfc)


def kernel(x, edge_index, W1, b1, W2, b2, Wfc, bfc):
    src = edge_index[0].astype(jnp.int32)
    dst = edge_index[1].astype(jnp.int32)
    pidx = jnp.arange(PAD, dtype=jnp.int32)
    srcP = jnp.concatenate([src, pidx % N]).reshape(E_PAD // CH, CH)
    dstP = jnp.concatenate([dst, N + (pidx % N_DUMMY)]).reshape(E_PAD // CH, CH)
    W2p = jnp.pad(W2, ((0, 0), (0, L - W2.shape[1])))
    b2p = jnp.pad(b2, (0, L - b2.shape[0]))[None, :]
    Wfcp = jnp.pad(Wfc, ((0, L - Wfc.shape[0]), (0, 0)))

    pdeg = _sc_degree(dstP)
    hs1, dinv = _tc_k1(x, W1, pdeg)
    p1 = _sc_propagate(srcP, dstP, hs1, F)
    hs2 = _tc_k2(p1, hs1, dinv, b1[None, :], W2p)
    p2 = _sc_propagate16(srcP, dstP, hs2)
    return _tc_k3(p2, hs2, dinv, b2p, Wfcp, bfc[None, :])
